# Initial kernel scaffold; baseline (speedup 1.0000x reference)
import jax
import jax.numpy as jnp
from jax.experimental import pallas as pl


def kernel(x, pos_edge_index, neg_edge_index, W, b, gamma, beta):
    raise NotImplementedError("write your pallas kernel here")


# speedup vs baseline: 24.4381x; 24.4381x over previous
import functools

import jax
import jax.numpy as jnp
from jax import lax
from jax.experimental import pallas as pl
from jax.experimental.pallas import tpu as pltpu
from jax.experimental.pallas import tpu_sc as plsc

N_NODES = 10000
D = 128
E_POS = 160000

NC = 2
NS = 16
NW = NC * NS
BSZ = 128
NB = (E_POS + NW * BSZ - 1) // (NW * BSZ)
CAP = NW * NB * BSZ
SLAB = 640
NPAD = NS * SLAB

def _mesh():
    return plsc.VectorSubcoreMesh(
        core_axis_name="c", subcore_axis_name="s", num_cores=NC, num_subcores=NS
    )


def _deg_body(cols_hbm, out_hbm, cols_v, ones_v, zeros_v, deg_sh):
    cid = lax.axis_index("c")
    sid = lax.axis_index("s")
    wid = sid * NC + cid
    pltpu.sync_copy(cols_hbm.at[wid], cols_v)

    def _fill(i, carry):
        ones_v[i, :] = jnp.ones((16,), jnp.float32)
        zeros_v[i, :] = jnp.zeros((16,), jnp.float32)
        return carry

    lax.fori_loop(0, BSZ, _fill, 0)
    for z in range(SLAB // BSZ):
        pltpu.sync_copy(zeros_v, deg_sh.at[pl.ds(sid * SLAB + z * BSZ, BSZ)])
    plsc.subcore_barrier()
    for j in range(NB):
        pltpu.sync_copy(ones_v, deg_sh.at[cols_v.at[j]], add=True)
    plsc.subcore_barrier()
    pltpu.sync_copy(
        deg_sh.at[pl.ds(sid * SLAB, SLAB)],
        out_hbm.at[cid, pl.ds(sid * SLAB, SLAB)],
    )


@functools.lru_cache(maxsize=None)
def _deg_kernel():
    return pl.kernel(
        _deg_body,
        out_type=jax.ShapeDtypeStruct((NC, NPAD, 16), jnp.float32),
        mesh=_mesh(),
        scratch_types=[
            pltpu.VMEM((NB, BSZ), jnp.int32),
            pltpu.VMEM((BSZ, 16), jnp.float32),
            pltpu.VMEM((BSZ, 16), jnp.float32),
            pltpu.VMEM_SHARED((NPAD, 16), jnp.float32),
        ],
    )


def _mm_body(degp_ref, x_ref, w_ref, y_ref):
    deg = degp_ref[0, :, 0:1] + degp_ref[1, :, 0:1] + 1.0
    dis = lax.rsqrt(deg)
    y_ref[...] = dis * jnp.dot(
        x_ref[...], w_ref[...], preferred_element_type=jnp.float32
    )


def _mm_call(degp, x, W):
    blk = 1000
    g = N_NODES // blk
    return pl.pallas_call(
        _mm_body,
        grid=(g,),
        in_specs=[
            pl.BlockSpec((NC, blk, 16), lambda i: (0, i, 0)),
            pl.BlockSpec((blk, D), lambda i: (i, 0)),
            pl.BlockSpec((D, D), lambda i: (0, 0)),
        ],
        out_specs=pl.BlockSpec((blk, D), lambda i: (i, 0)),
        out_shape=jax.ShapeDtypeStruct((N_NODES, D), jnp.float32),
    )(degp, x, W)


def _agg_body(rows_hbm, cols_hbm, y_hbm, out_hbm,
              rows_v, cols_v, buf0, buf1, agg_sh, sem0, sem1):
    cid = lax.axis_index("c")
    sid = lax.axis_index("s")
    wid = sid * NC + cid
    pltpu.sync_copy(rows_hbm.at[wid], rows_v)
    pltpu.sync_copy(cols_hbm.at[wid], cols_v)

    def _zero(i, carry):
        for k in range(D // 16):
            buf0[i, 16 * k:16 * (k + 1)] = jnp.zeros((16,), jnp.float32)
        return carry

    lax.fori_loop(0, BSZ, _zero, 0)
    for z in range(SLAB // BSZ):
        pltpu.sync_copy(buf0, agg_sh.at[pl.ds(sid * SLAB + z * BSZ, BSZ)])
    plsc.subcore_barrier()

    bufs = (buf0, buf1)
    sems = (sem0, sem1)
    descs = [None, None]
    descs[0] = pltpu.async_copy(y_hbm.at[rows_v.at[0]], buf0, sem0)
    for j in range(NB):
        cur = j & 1
        nxt = (j + 1) & 1
        if j + 1 < NB:
            descs[nxt] = pltpu.async_copy(
                y_hbm.at[rows_v.at[j + 1]], bufs[nxt], sems[nxt]
            )
        descs[cur].wait()
        pltpu.sync_copy(bufs[cur], agg_sh.at[cols_v.at[j]], add=True)
    plsc.subcore_barrier()
    for z in range(SLAB // BSZ):
        off = sid * SLAB + z * BSZ
        pltpu.sync_copy(agg_sh.at[pl.ds(off, BSZ)], out_hbm.at[cid, pl.ds(off, BSZ)])


@functools.lru_cache(maxsize=None)
def _agg_kernel():
    return pl.kernel(
        _agg_body,
        out_type=jax.ShapeDtypeStruct((NC, NPAD, D), jnp.float32),
        mesh=_mesh(),
        scratch_types=[
            pltpu.VMEM((NB, BSZ), jnp.int32),
            pltpu.VMEM((NB, BSZ), jnp.int32),
            pltpu.VMEM((BSZ, D), jnp.float32),
            pltpu.VMEM((BSZ, D), jnp.float32),
            pltpu.VMEM_SHARED((NPAD, D), jnp.float32),
            pltpu.SemaphoreType.DMA,
            pltpu.SemaphoreType.DMA,
        ],
    )


def _stat_body(aggp_ref, y_ref, degp_ref, b_ref, t_ref, s1_ref, s2_ref):
    i = pl.program_id(0)
    deg = degp_ref[0, :, 0:1] + degp_ref[1, :, 0:1] + 1.0
    dis = lax.rsqrt(deg)
    t = dis * (aggp_ref[0] + aggp_ref[1] + y_ref[...]) + b_ref[...]
    t_ref[...] = t

    @pl.when(i == 0)
    def _():
        s1_ref[...] = jnp.zeros_like(s1_ref)
        s2_ref[...] = jnp.zeros_like(s2_ref)

    s1_ref[...] += jnp.sum(t, axis=0, keepdims=True)
    s2_ref[...] += jnp.sum(t * t, axis=0, keepdims=True)


def _stat_call(aggp, y, degp, b2):
    blk = 1000
    g = N_NODES // blk
    return pl.pallas_call(
        _stat_body,
        grid=(g,),
        in_specs=[
            pl.BlockSpec((NC, blk, D), lambda i: (0, i, 0)),
            pl.BlockSpec((blk, D), lambda i: (i, 0)),
            pl.BlockSpec((NC, blk, 16), lambda i: (0, i, 0)),
            pl.BlockSpec((1, D), lambda i: (0, 0)),
        ],
        out_specs=[
            pl.BlockSpec((blk, D), lambda i: (i, 0)),
            pl.BlockSpec((1, D), lambda i: (0, 0)),
            pl.BlockSpec((1, D), lambda i: (0, 0)),
        ],
        out_shape=[
            jax.ShapeDtypeStruct((N_NODES, D), jnp.float32),
            jax.ShapeDtypeStruct((1, D), jnp.float32),
            jax.ShapeDtypeStruct((1, D), jnp.float32),
        ],
    )(aggp, y, degp, b2)


def _bn_body(t_ref, s1_ref, s2_ref, g_ref, be_ref, o_ref):
    n = jnp.float32(N_NODES)
    mean = s1_ref[...] / n
    var = s2_ref[...] / n - mean * mean
    scale = g_ref[...] * lax.rsqrt(var + 1e-5)
    shift = be_ref[...] - mean * scale
    o_ref[...] = jnp.maximum(t_ref[...] * scale + shift, 0.0)


def _bn_call(t, s1, s2, g2, be2):
    blk = 1000
    g = N_NODES // blk
    return pl.pallas_call(
        _bn_body,
        grid=(g,),
        in_specs=[
            pl.BlockSpec((blk, D), lambda i: (i, 0)),
            pl.BlockSpec((1, D), lambda i: (0, 0)),
            pl.BlockSpec((1, D), lambda i: (0, 0)),
            pl.BlockSpec((1, D), lambda i: (0, 0)),
            pl.BlockSpec((1, D), lambda i: (0, 0)),
        ],
        out_specs=pl.BlockSpec((blk, D), lambda i: (i, 0)),
        out_shape=jax.ShapeDtypeStruct((N_NODES, D), jnp.float32),
    )(t, s1, s2, g2, be2)


def kernel(x, pos_edge_index, neg_edge_index, W, b, gamma, beta):
    del neg_edge_index
    row = pos_edge_index[0].astype(jnp.int32)
    col = pos_edge_index[1].astype(jnp.int32)
    pad = CAP - E_POS
    rows_r = jnp.concatenate([row, jnp.zeros((pad,), jnp.int32)]).reshape(NW, NB, BSZ)
    cols_r = jnp.concatenate(
        [col, jnp.full((pad,), N_NODES, jnp.int32)]
    ).reshape(NW, NB, BSZ)

    degp = _deg_kernel()(cols_r)
    y = _mm_call(degp, x, W)
    aggp = _agg_kernel()(rows_r, cols_r, y)
    t, s1, s2 = _stat_call(aggp, y, degp, b.reshape(1, D))
    return _bn_call(t, s1, s2, gamma.reshape(1, D), beta.reshape(1, D))
